# baseline (speedup 1.0000x reference)
import functools

import jax
import jax.numpy as jnp
import numpy as np
from jax.experimental import pallas as pl
from jax.experimental.pallas import tpu as pltpu

_N_RBF = 16
_CUTOFF = 5.0
_R0 = 0.5

_W_COEF = [
    0.999999599920672, -0.09869489087930657, 0.0032464293919966616,
    -4.263542181345002e-05, 2.939347038148025e-07, -1.053271470369752e-09,
]


def _silu(x):
    return x * jax.nn.sigmoid(x)


def _body(pos_ref, posT_ref, w1t_ref, b1_ref, w2t_ref, b2_ref, w3t_ref,
          b3_ref, out_ref, feat_ref, *, br, n):
    bi = pl.program_id(0)
    bj = pl.program_id(1)
    nb = n // br

    @pl.when(jnp.logical_and(bi == 0, bj == 0))
    def _():
        feat_ref[...] = jnp.zeros((n, _N_RBF), jnp.float32)

    @pl.when(bj >= bi)
    def _():
        pos = pos_ref[...]
        posT = posT_ref[...]

        d2 = jnp.full((br, br), 1e-12, jnp.float32)
        for c in range(3):
            diff = pos[:, c:c + 1] - posT[c:c + 1, :]
            d2 = d2 + diff * diff

        r = jax.lax.rsqrt(d2)
        r = r * (1.5 - (0.5 * d2) * (r * r))
        dist = d2 * r

        mask = (d2 > 1.5e-12) & (d2 < _CUTOFF * _CUTOFF)

        w = jnp.full((br, br), np.float32(_W_COEF[-1]), jnp.float32)
        for cc in _W_COEF[-2::-1]:
            w = w * d2 + np.float32(cc)
        w = jnp.where(mask, jnp.maximum(w, 0.0), 0.0)

        eta = 0.5 * _CUTOFF / _N_RBF
        inv2eta2 = 1.0 / (2.0 * eta * eta)
        inveta2 = 1.0 / (eta * eta)
        log2e = float(np.log2(np.e))
        centers = np.linspace(_R0, _CUTOFF, _N_RBF)
        dc = float(centers[1] - centers[0])

        lw = jnp.log2(w)
        g2 = lw + (dist - np.float32(centers[0])) ** 2 * \
            np.float32(-inv2eta2 * log2e)
        step = dist * np.float32(dc * inveta2 * log2e) + \
            np.float32(-(centers[1] ** 2 - centers[0] ** 2) * inv2eta2 * log2e)
        step2 = np.float32(-2.0 * dc * dc * inv2eta2 * log2e)
        rows = []
        cols = []
        for k in range(_N_RBF):
            wek = jnp.exp2(g2)
            rows.append(jnp.sum(wek, axis=1, keepdims=True))
            cols.append(jnp.sum(wek, axis=0, keepdims=True))
            if k + 1 < _N_RBF:
                g2 = g2 + step
                step = step + step2
        rowsum = jnp.concatenate(rows, axis=1)
        feat_ref[pl.ds(bi * br, br), :] += rowsum

        @pl.when(bj > bi)
        def _():
            colsum = jnp.concatenate(cols, axis=0).T
            feat_ref[pl.ds(bj * br, br), :] += colsum

    @pl.when(jnp.logical_and(bi == nb - 1, bj == nb - 1))
    def _():
        feats = feat_ref[...]
        h = _silu(jnp.dot(feats, w1t_ref[...],
                          preferred_element_type=jnp.float32) + b1_ref[...])
        h = _silu(jnp.dot(h, w2t_ref[...],
                          preferred_element_type=jnp.float32) + b2_ref[...])
        o = jnp.dot(h, w3t_ref[...], preferred_element_type=jnp.float32)
        total = jnp.sum(o) + np.float32(n) * b3_ref[0, 0]
        out_ref[...] = jnp.full((1, 1), total, jnp.float32)


@jax.jit
def _run(positions, W1, b1, W2, b2, W3, b3):
    n = positions.shape[0]
    br = 256
    nb = n // br
    posT = positions.T
    w1t = W1.T
    w2t = W2.T
    w3t = W3.T
    b1r = b1.reshape(1, -1)
    b2r = b2.reshape(1, -1)
    b3r = b3.reshape(1, 1)

    out = pl.pallas_call(
        functools.partial(_body, br=br, n=n),
        grid=(nb, nb),
        in_specs=[
            pl.BlockSpec((br, 3), lambda i, j: (i, 0)),
            pl.BlockSpec((3, br), lambda i, j: (0, j)),
            pl.BlockSpec(w1t.shape, lambda i, j: (0, 0)),
            pl.BlockSpec(b1r.shape, lambda i, j: (0, 0)),
            pl.BlockSpec(w2t.shape, lambda i, j: (0, 0)),
            pl.BlockSpec(b2r.shape, lambda i, j: (0, 0)),
            pl.BlockSpec(w3t.shape, lambda i, j: (0, 0)),
            pl.BlockSpec(b3r.shape, lambda i, j: (0, 0)),
        ],
        out_specs=pl.BlockSpec((1, 1), lambda i, j: (0, 0)),
        out_shape=jax.ShapeDtypeStruct((1, 1), jnp.float32),
        scratch_shapes=[pltpu.VMEM((n, _N_RBF), jnp.float32)],
        compiler_params=pltpu.CompilerParams(
            dimension_semantics=("arbitrary", "arbitrary"),
        ),
    )(positions, posT, w1t, b1r, w2t, b2r, w3t, b3r)
    return out[0, 0]


def kernel(positions, W1, b1, W2, b2, W3, b3):
    return _run(positions, W1, b1, W2, b2, W3, b3)


# speedup vs baseline: 1.1081x; 1.1081x over previous
import functools

import jax
import jax.numpy as jnp
import numpy as np
from jax.experimental import pallas as pl
from jax.experimental.pallas import tpu as pltpu

_N_RBF = 16
_CUTOFF = 5.0
_R0 = 0.5

_W_COEF = [
    0.999999599920672, -0.09869489087930657, 0.0032464293919966616,
    -4.263542181345002e-05, 2.939347038148025e-07, -1.053271470369752e-09,
]


def _silu(x):
    return x * jax.nn.sigmoid(x)


def _body(bi_ref, bj_ref, pos_ref, posT_ref, w1t_ref, b1_ref, w2t_ref,
          b2_ref, w3t_ref, b3_ref, out_ref, feat_ref, *, br, n, steps):
    t = pl.program_id(0)
    bi = bi_ref[t]
    bj = bj_ref[t]

    @pl.when(t == 0)
    def _():
        feat_ref[...] = jnp.zeros((n, _N_RBF), jnp.float32)

    pos = pos_ref[...]
    posT = posT_ref[...]

    d2 = jnp.full((br, br), 1e-12, jnp.float32)
    for c in range(3):
        diff = pos[:, c:c + 1] - posT[c:c + 1, :]
        d2 = d2 + diff * diff

    r = jax.lax.rsqrt(d2)
    r = r * (1.5 - (0.5 * d2) * (r * r))
    dist = d2 * r

    mask = (d2 > 1.5e-12) & (d2 < _CUTOFF * _CUTOFF)

    w = jnp.full((br, br), np.float32(_W_COEF[-1]), jnp.float32)
    for cc in _W_COEF[-2::-1]:
        w = w * d2 + np.float32(cc)
    w = jnp.where(mask, jnp.maximum(w, 0.0), 0.0)

    eta = 0.5 * _CUTOFF / _N_RBF
    inv2eta2 = 1.0 / (2.0 * eta * eta)
    inveta2 = 1.0 / (eta * eta)
    log2e = float(np.log2(np.e))
    centers = np.linspace(_R0, _CUTOFF, _N_RBF)
    dc = float(centers[1] - centers[0])

    lw = jnp.log2(w)
    g2 = lw + (dist - np.float32(centers[0])) ** 2 * \
        np.float32(-inv2eta2 * log2e)
    step = dist * np.float32(dc * inveta2 * log2e) + \
        np.float32(-(centers[1] ** 2 - centers[0] ** 2) * inv2eta2 * log2e)
    step2 = np.float32(-2.0 * dc * dc * inv2eta2 * log2e)
    weks = []
    rows = []
    for k in range(_N_RBF):
        wek = jnp.exp2(g2)
        weks.append(wek)
        rows.append(jnp.sum(wek, axis=1, keepdims=True))
        if k + 1 < _N_RBF:
            g2 = g2 + step
            step = step + step2
    rowsum = jnp.concatenate(rows, axis=1)
    feat_ref[pl.ds(bi * br, br), :] += rowsum

    @pl.when(bj != bi)
    def _():
        cols = [jnp.sum(wk, axis=0, keepdims=True) for wk in weks]
        colsum = jnp.concatenate(cols, axis=0).T
        feat_ref[pl.ds(bj * br, br), :] += colsum

    @pl.when(t == steps - 1)
    def _():
        feats = feat_ref[...]
        h = _silu(jnp.dot(feats, w1t_ref[...],
                          preferred_element_type=jnp.float32) + b1_ref[...])
        h = _silu(jnp.dot(h, w2t_ref[...],
                          preferred_element_type=jnp.float32) + b2_ref[...])
        o = jnp.dot(h, w3t_ref[...], preferred_element_type=jnp.float32)
        total = jnp.sum(o) + np.float32(n) * b3_ref[0, 0]
        out_ref[...] = jnp.full((1, 1), total, jnp.float32)


@jax.jit
def _run(positions, W1, b1, W2, b2, W3, b3):
    n = positions.shape[0]
    br = 512
    nb = n // br
    posT = positions.T
    w1t = W1.T
    w2t = W2.T
    w3t = W3.T
    b1r = b1.reshape(1, -1)
    b2r = b2.reshape(1, -1)
    b3r = b3.reshape(1, 1)

    pairs = [(i, j) for i in range(nb) for j in range(i, nb)]
    steps = len(pairs)
    bi_map = jnp.asarray([p[0] for p in pairs], jnp.int32)
    bj_map = jnp.asarray([p[1] for p in pairs], jnp.int32)

    grid_spec = pltpu.PrefetchScalarGridSpec(
        num_scalar_prefetch=2,
        grid=(steps,),
        in_specs=[
            pl.BlockSpec((br, 3), lambda t, bi, bj: (bi[t], 0)),
            pl.BlockSpec((3, br), lambda t, bi, bj: (0, bj[t])),
            pl.BlockSpec(w1t.shape, lambda t, bi, bj: (0, 0)),
            pl.BlockSpec(b1r.shape, lambda t, bi, bj: (0, 0)),
            pl.BlockSpec(w2t.shape, lambda t, bi, bj: (0, 0)),
            pl.BlockSpec(b2r.shape, lambda t, bi, bj: (0, 0)),
            pl.BlockSpec(w3t.shape, lambda t, bi, bj: (0, 0)),
            pl.BlockSpec(b3r.shape, lambda t, bi, bj: (0, 0)),
        ],
        out_specs=pl.BlockSpec((1, 1), lambda t, bi, bj: (0, 0)),
        scratch_shapes=[pltpu.VMEM((n, _N_RBF), jnp.float32)],
    )

    out = pl.pallas_call(
        functools.partial(_body, br=br, n=n, steps=steps),
        grid_spec=grid_spec,
        out_shape=jax.ShapeDtypeStruct((1, 1), jnp.float32),
        compiler_params=pltpu.CompilerParams(
            dimension_semantics=("arbitrary",),
        ),
    )(bi_map, bj_map, positions, posT, w1t, b1r, w2t, b2r, w3t, b3r)
    return out[0, 0]


def kernel(positions, W1, b1, W2, b2, W3, b3):
    return _run(positions, W1, b1, W2, b2, W3, b3)


# speedup vs baseline: 1.2656x; 1.1421x over previous
import functools

import jax
import jax.numpy as jnp
import numpy as np
from jax.experimental import pallas as pl
from jax.experimental.pallas import tpu as pltpu

_N_RBF = 16
_CUTOFF = 5.0
_R0 = 0.5

_W_COEF = [
    0.999999599920672, -0.09869489087930657, 0.0032464293919966616,
    -4.263542181345002e-05, 2.939347038148025e-07, -1.053271470369752e-09,
]


def _silu(x):
    return x * jax.nn.sigmoid(x)


def _body(bi_ref, bj_ref, pos_ref, posT_ref, w1t_ref, b1_ref, w2t_ref,
          b2_ref, w3t_ref, b3_ref, out_ref, feat_ref, *, br, n, steps):
    t = pl.program_id(0)
    bi = bi_ref[t]
    bj = bj_ref[t]

    @pl.when(t == 0)
    def _():
        feat_ref[...] = jnp.zeros((n, _N_RBF), jnp.float32)

    pos = pos_ref[...]
    posT = posT_ref[...]

    d2 = jnp.full((br, br), 1e-12, jnp.float32)
    for c in range(3):
        diff = pos[:, c:c + 1] - posT[c:c + 1, :]
        d2 = d2 + diff * diff

    r = jax.lax.rsqrt(d2)
    r = r * (1.5 - (0.5 * d2) * (r * r))
    dist = d2 * r

    mask = (d2 > 1.5e-12) & (d2 < _CUTOFF * _CUTOFF)

    w = jnp.full((br, br), np.float32(_W_COEF[-1]), jnp.float32)
    for cc in _W_COEF[-2::-1]:
        w = w * d2 + np.float32(cc)
    w = jnp.where(mask, jnp.maximum(w, 0.0), 0.0)

    eta = 0.5 * _CUTOFF / _N_RBF
    inv2eta2 = 1.0 / (2.0 * eta * eta)
    inveta2 = 1.0 / (eta * eta)
    log2e = float(np.log2(np.e))
    centers = np.linspace(_R0, _CUTOFF, _N_RBF)
    dc = float(centers[1] - centers[0])

    lw = jnp.log2(w)
    g2 = lw + (dist - np.float32(centers[0])) ** 2 * \
        np.float32(-inv2eta2 * log2e)
    step = dist * np.float32(dc * inveta2 * log2e) + \
        np.float32(-(centers[1] ** 2 - centers[0] ** 2) * inv2eta2 * log2e)
    step2 = np.float32(-2.0 * dc * dc * inv2eta2 * log2e)
    rows = []
    cols = []
    for k in range(_N_RBF):
        wek = jnp.exp2(g2)
        rows.append(jnp.sum(wek, axis=1, keepdims=True))
        cols.append(jnp.sum(wek, axis=0, keepdims=True))
        if k + 1 < _N_RBF:
            g2 = g2 + step
            step = step + step2
    rowsum = jnp.concatenate(rows, axis=1)
    feat_ref[pl.ds(bi * br, br), :] += rowsum

    @pl.when(bj != bi)
    def _():
        colsum = jnp.concatenate(cols, axis=0).T
        feat_ref[pl.ds(bj * br, br), :] += colsum

    @pl.when(t == steps - 1)
    def _():
        feats = feat_ref[...]
        h = _silu(jnp.dot(feats, w1t_ref[...],
                          preferred_element_type=jnp.float32) + b1_ref[...])
        h = _silu(jnp.dot(h, w2t_ref[...],
                          preferred_element_type=jnp.float32) + b2_ref[...])
        o = jnp.dot(h, w3t_ref[...], preferred_element_type=jnp.float32)
        total = jnp.sum(o) + np.float32(n) * b3_ref[0, 0]
        out_ref[...] = jnp.full((1, 1), total, jnp.float32)


@jax.jit
def _run(positions, W1, b1, W2, b2, W3, b3):
    n = positions.shape[0]
    br = 512
    nb = n // br
    posT = positions.T
    w1t = W1.T
    w2t = W2.T
    w3t = W3.T
    b1r = b1.reshape(1, -1)
    b2r = b2.reshape(1, -1)
    b3r = b3.reshape(1, 1)

    pairs = [(i, j) for i in range(nb) for j in range(i, nb)]
    steps = len(pairs)
    bi_map = jnp.asarray([p[0] for p in pairs], jnp.int32)
    bj_map = jnp.asarray([p[1] for p in pairs], jnp.int32)

    grid_spec = pltpu.PrefetchScalarGridSpec(
        num_scalar_prefetch=2,
        grid=(steps,),
        in_specs=[
            pl.BlockSpec((br, 3), lambda t, bi, bj: (bi[t], 0)),
            pl.BlockSpec((3, br), lambda t, bi, bj: (0, bj[t])),
            pl.BlockSpec(w1t.shape, lambda t, bi, bj: (0, 0)),
            pl.BlockSpec(b1r.shape, lambda t, bi, bj: (0, 0)),
            pl.BlockSpec(w2t.shape, lambda t, bi, bj: (0, 0)),
            pl.BlockSpec(b2r.shape, lambda t, bi, bj: (0, 0)),
            pl.BlockSpec(w3t.shape, lambda t, bi, bj: (0, 0)),
            pl.BlockSpec(b3r.shape, lambda t, bi, bj: (0, 0)),
        ],
        out_specs=pl.BlockSpec((1, 1), lambda t, bi, bj: (0, 0)),
        scratch_shapes=[pltpu.VMEM((n, _N_RBF), jnp.float32)],
    )

    out = pl.pallas_call(
        functools.partial(_body, br=br, n=n, steps=steps),
        grid_spec=grid_spec,
        out_shape=jax.ShapeDtypeStruct((1, 1), jnp.float32),
        compiler_params=pltpu.CompilerParams(
            dimension_semantics=("arbitrary",),
        ),
    )(bi_map, bj_map, positions, posT, w1t, b1r, w2t, b2r, w3t, b3r)
    return out[0, 0]


def kernel(positions, W1, b1, W2, b2, W3, b3):
    return _run(positions, W1, b1, W2, b2, W3, b3)


# speedup vs baseline: 1.3058x; 1.0318x over previous
import functools

import jax
import jax.numpy as jnp
import numpy as np
from jax.experimental import pallas as pl
from jax.experimental.pallas import tpu as pltpu

_N_RBF = 16
_CUTOFF = 5.0
_R0 = 0.5

_W_COEF = [
    0.999999599920672, -0.09869489087930657, 0.0032464293919966616,
    -4.263542181345002e-05, 2.939347038148025e-07, -1.053271470369752e-09,
]


def _silu(x):
    return x * jax.nn.sigmoid(x)


def _body(bi_ref, bj_ref, pos_ref, posT_ref, w1t_ref, b1_ref, w2t_ref,
          b2_ref, w3t_ref, b3_ref, out_ref, feat_ref, *, br, n, steps):
    t = pl.program_id(0)
    bi = bi_ref[t]
    bj = bj_ref[t]

    @pl.when(t == 0)
    def _():
        feat_ref[...] = jnp.zeros((n, _N_RBF), jnp.float32)

    pos = pos_ref[...]
    posT = posT_ref[...]

    d2 = jnp.full((br, br), 1e-12, jnp.float32)
    for c in range(3):
        diff = pos[:, c:c + 1] - posT[c:c + 1, :]
        d2 = d2 + diff * diff

    r = jax.lax.rsqrt(d2)
    r = r * (1.5 - (0.5 * d2) * (r * r))
    dist = d2 * r

    mask = (d2 > 1.5e-12) & (d2 < _CUTOFF * _CUTOFF)

    w = jnp.full((br, br), np.float32(_W_COEF[-1]), jnp.float32)
    for cc in _W_COEF[-2::-1]:
        w = w * d2 + np.float32(cc)
    w = jnp.where(mask, jnp.maximum(w, 0.0), 0.0)

    eta = 0.5 * _CUTOFF / _N_RBF
    inv2eta2 = 1.0 / (2.0 * eta * eta)
    inveta2 = 1.0 / (eta * eta)
    log2e = float(np.log2(np.e))
    centers = np.linspace(_R0, _CUTOFF, _N_RBF)
    dc = float(centers[1] - centers[0])

    dist5 = jnp.minimum(dist, _CUTOFF)
    lw = jnp.log2(w)
    g2 = lw + (dist5 - np.float32(centers[0])) ** 2 * \
        np.float32(-inv2eta2 * log2e)
    step = dist5 * np.float32(dc * inveta2 * log2e) + \
        np.float32(-(centers[1] ** 2 - centers[0] ** 2) * inv2eta2 * log2e)
    s2 = -dc * dc * inveta2 * log2e
    qstep = step * np.float32(4.0) + np.float32(6.0 * s2)
    qstep2 = np.float32(16.0 * s2)
    m = jnp.exp2(step)
    mc1 = m * np.float32(2.0 ** s2)
    mc2 = mc1 * np.float32(2.0 ** s2)
    c4 = np.float32(2.0 ** (4.0 * s2))
    rows = []
    cols = []
    for q in range(_N_RBF // 4):
        wek = jnp.exp2(g2)
        for j in range(4):
            rows.append(jnp.sum(wek, axis=1, keepdims=True))
            cols.append(jnp.sum(wek, axis=0, keepdims=True))
            if j == 0:
                wek = wek * m
            elif j == 1:
                wek = wek * mc1
            elif j == 2:
                wek = wek * mc2
        if q + 1 < _N_RBF // 4:
            g2 = g2 + qstep
            qstep = qstep + qstep2
            m = m * c4
            mc1 = mc1 * c4
            mc2 = mc2 * c4
    rowsum = jnp.concatenate(rows, axis=1)
    feat_ref[pl.ds(bi * br, br), :] += rowsum

    @pl.when(bj != bi)
    def _():
        colsum = jnp.concatenate(cols, axis=0).T
        feat_ref[pl.ds(bj * br, br), :] += colsum

    @pl.when(t == steps - 1)
    def _():
        feats = feat_ref[...]
        h = _silu(jnp.dot(feats, w1t_ref[...],
                          preferred_element_type=jnp.float32) + b1_ref[...])
        h = _silu(jnp.dot(h, w2t_ref[...],
                          preferred_element_type=jnp.float32) + b2_ref[...])
        o = jnp.dot(h, w3t_ref[...], preferred_element_type=jnp.float32)
        total = jnp.sum(o) + np.float32(n) * b3_ref[0, 0]
        out_ref[...] = jnp.full((1, 1), total, jnp.float32)


@jax.jit
def _run(positions, W1, b1, W2, b2, W3, b3):
    n = positions.shape[0]
    br = 512
    nb = n // br
    posT = positions.T
    w1t = W1.T
    w2t = W2.T
    w3t = W3.T
    b1r = b1.reshape(1, -1)
    b2r = b2.reshape(1, -1)
    b3r = b3.reshape(1, 1)

    pairs = [(i, j) for i in range(nb) for j in range(i, nb)]
    steps = len(pairs)
    bi_map = jnp.asarray([p[0] for p in pairs], jnp.int32)
    bj_map = jnp.asarray([p[1] for p in pairs], jnp.int32)

    grid_spec = pltpu.PrefetchScalarGridSpec(
        num_scalar_prefetch=2,
        grid=(steps,),
        in_specs=[
            pl.BlockSpec((br, 3), lambda t, bi, bj: (bi[t], 0)),
            pl.BlockSpec((3, br), lambda t, bi, bj: (0, bj[t])),
            pl.BlockSpec(w1t.shape, lambda t, bi, bj: (0, 0)),
            pl.BlockSpec(b1r.shape, lambda t, bi, bj: (0, 0)),
            pl.BlockSpec(w2t.shape, lambda t, bi, bj: (0, 0)),
            pl.BlockSpec(b2r.shape, lambda t, bi, bj: (0, 0)),
            pl.BlockSpec(w3t.shape, lambda t, bi, bj: (0, 0)),
            pl.BlockSpec(b3r.shape, lambda t, bi, bj: (0, 0)),
        ],
        out_specs=pl.BlockSpec((1, 1), lambda t, bi, bj: (0, 0)),
        scratch_shapes=[pltpu.VMEM((n, _N_RBF), jnp.float32)],
    )

    out = pl.pallas_call(
        functools.partial(_body, br=br, n=n, steps=steps),
        grid_spec=grid_spec,
        out_shape=jax.ShapeDtypeStruct((1, 1), jnp.float32),
        compiler_params=pltpu.CompilerParams(
            dimension_semantics=("arbitrary",),
        ),
    )(bi_map, bj_map, positions, posT, w1t, b1r, w2t, b2r, w3t, b3r)
    return out[0, 0]


def kernel(positions, W1, b1, W2, b2, W3, b3):
    return _run(positions, W1, b1, W2, b2, W3, b3)
